# baseline (device time: 1496066 ns/iter reference)
import jax
import jax.numpy as jnp
from jax import lax
from jax.experimental import pallas as pl
from jax.experimental.pallas import tpu as pltpu

N_DEV = 32
N_RINGS = 8


def _gelu(y):
    c = 0.7978845608028654
    return 0.5 * y * (1.0 + jnp.tanh(c * (y + 0.044715 * y * y * y)))


def kernel(x, w_mat):
    m, _ = x.shape
    _, n = w_mat.shape
    chunk = m // N_DEV
    q = n // N_RINGS

    n_steps = 2 * (N_DEV - 1)

    half = N_RINGS // 2
    order = [r for pair in zip(range(half), range(half, N_RINGS))
             for r in pair]
    rings = [(r, r * q, r < half) for r in order]

    def body(x_ref, w_ref, out_ref, comm, acc,
             send_sems, recv_sems, credit_sems, out_sems):
        def k_of_pos(p):
            z = p // 8
            w = lax.rem(p, 8)
            y = w // 2
            xb = lax.rem(w, 2)
            ye = lax.rem(y, 2)
            xc = jnp.where(ye == 0, xb, 1 - xb)
            idx0 = y * 4 + jnp.where(ye == 0, z, 3 - z)
            return jnp.where(xc == 0, idx0, N_DEV - 1 - idx0)

        def pos_of_k(k):
            k = lax.rem(k + 2 * N_DEV, N_DEV)
            xc = jnp.where(k < 16, 0, 1)
            idx0 = jnp.where(k < 16, k, N_DEV - 1 - k)
            y = idx0 // 4
            zz = lax.rem(idx0, 4)
            ye = lax.rem(y, 2)
            z = jnp.where(ye == 0, zz, 3 - zz)
            xb = jnp.where(ye == 0, xc, 1 - xc)
            return z * 8 + y * 2 + xb

        me = k_of_pos(lax.axis_index("i"))
        left = pos_of_k(me + N_DEV - 1)
        right = pos_of_k(me + 1)

        def dst_dev(fwd):
            return (right,) if fwd else (left,)

        def upstream(fwd):
            return (left,) if fwd else (right,)

        def arrival_chunk(s, fwd):
            rs = lax.rem(me - s - 1 + 3 * N_DEV, N_DEV)
            ag = lax.rem(me - (s - (N_DEV - 1)) + 3 * N_DEV, N_DEV)
            if not fwd:
                rs = lax.rem(me + s + 1, N_DEV)
                ag = lax.rem(me + s - (N_DEV - 1) + 3 * N_DEV, N_DEV)
            return jnp.where(s < N_DEV - 1, rs, ag)

        def partial(c, col0):
            xs = x_ref[pl.ds(pos_of_k(c) * chunk, chunk), :]
            return jnp.dot(xs, w_ref[:, col0:col0 + q],
                           preferred_element_type=jnp.float32)

        def desc(r, fwd, slot_send, slot_recv):
            return pltpu.make_async_remote_copy(
                src_ref=comm.at[r, slot_send],
                dst_ref=comm.at[r, slot_recv],
                send_sem=send_sems.at[r, slot_send],
                recv_sem=recv_sems.at[r, slot_recv],
                device_id=dst_dev(fwd),
                device_id_type=pl.DeviceIdType.MESH,
            )

        def signal_credit(r, fwd):
            pl.semaphore_signal(credit_sems.at[r], inc=1,
                                device_id=upstream(fwd),
                                device_id_type=pl.DeviceIdType.MESH)

        def start_next(r, fwd, slot_send, slot_recv):
            pl.semaphore_wait(credit_sems.at[r], 1)
            desc(r, fwd, slot_recv, slot_send).start()

        def out_cp(r, slot, c, col0):
            return pltpu.make_async_copy(
                comm.at[r, slot],
                out_ref.at[pl.ds(pos_of_k(c) * chunk, chunk),
                           pl.ds(col0, q)],
                out_sems.at[r, slot],
            )

        barrier = pltpu.get_barrier_semaphore()
        pl.semaphore_signal(barrier, inc=1, device_id=(left,),
                            device_id_type=pl.DeviceIdType.MESH)
        pl.semaphore_signal(barrier, inc=1, device_id=(right,),
                            device_id_type=pl.DeviceIdType.MESH)
        pl.semaphore_wait(barrier, 2)

        for r, col0, fwd in rings:
            comm[r, 0] = partial(me, col0)
            desc(r, fwd, 0, 1).start()
        for r, col0, fwd in rings:
            acc[r] = partial(arrival_chunk(jnp.int32(0), fwd), col0)

        def rs_step(s, carry):
            slot_send = lax.rem(s, 2)
            slot_recv = 1 - slot_send
            for r, col0, fwd in rings:
                d = desc(r, fwd, slot_send, slot_recv)
                d.wait_recv()
                comm[r, slot_recv] = comm[r, slot_recv] + acc[r]
                d.wait_send()
                signal_credit(r, fwd)
                start_next(r, fwd, slot_send, slot_recv)
                acc[r] = partial(arrival_chunk(s + 1, fwd), col0)
            return carry

        lax.fori_loop(0, N_DEV - 2, rs_step, 0, unroll=False)

        slot_send = (N_DEV - 2) % 2
        slot_recv = 1 - slot_send
        s30 = jnp.int32(N_DEV - 2)
        for r, col0, fwd in rings:
            d = desc(r, fwd, slot_send, slot_recv)
            d.wait_recv()
            comm[r, slot_recv] = _gelu(comm[r, slot_recv] + acc[r])
            d.wait_send()
            signal_credit(r, fwd)
            start_next(r, fwd, slot_send, slot_recv)
            out_cp(r, slot_recv, arrival_chunk(s30, fwd), col0).start()

        def ag_step(s, carry):
            slot_send = lax.rem(s, 2)
            slot_recv = 1 - slot_send
            for r, col0, fwd in rings:
                d = desc(r, fwd, slot_send, slot_recv)
                d.wait_recv()
                d.wait_send()
                out_cp(r, slot_send, arrival_chunk(s - 1, fwd),
                       col0).wait()
                signal_credit(r, fwd)
                start_next(r, fwd, slot_send, slot_recv)
                out_cp(r, slot_recv, arrival_chunk(s, fwd), col0).start()
            return carry

        lax.fori_loop(N_DEV - 1, n_steps - 1, ag_step, 0, unroll=False)

        s_last = jnp.int32(n_steps - 1)
        slot_send = (n_steps - 1) % 2
        slot_recv = 1 - slot_send
        for r, col0, fwd in rings:
            d = desc(r, fwd, slot_send, slot_recv)
            d.wait_recv()
            out_cp(r, slot_recv, arrival_chunk(s_last, fwd), col0).start()
            d.wait_send()
            out_cp(r, slot_send, arrival_chunk(s_last - 1, fwd),
                   col0).wait()
            out_cp(r, slot_recv, arrival_chunk(s_last, fwd), col0).wait()

    return pl.pallas_call(
        body,
        out_shape=jax.ShapeDtypeStruct((m, n), jnp.float32),
        in_specs=[
            pl.BlockSpec(memory_space=pltpu.VMEM),
            pl.BlockSpec(memory_space=pltpu.VMEM),
        ],
        out_specs=pl.BlockSpec(memory_space=pl.ANY),
        scratch_shapes=[
            pltpu.VMEM((N_RINGS, 2, chunk, q), jnp.float32),
            pltpu.VMEM((N_RINGS, chunk, q), jnp.float32),
            pltpu.SemaphoreType.DMA((N_RINGS, 2)),
            pltpu.SemaphoreType.DMA((N_RINGS, 2)),
            pltpu.SemaphoreType.REGULAR((N_RINGS,)),
            pltpu.SemaphoreType.DMA((N_RINGS, 2)),
        ],
        compiler_params=pltpu.CompilerParams(collective_id=0),
    )(x, w_mat)
